# baseline (speedup 1.0000x reference)
import jax
import jax.numpy as jnp
from jax.experimental import pallas as pl

NUM_USER = 50000
NUM_ITEM = 50000
EMBED_DIM = 64
NUM_LAYER = 3
BATCH = 4096
N_NEG = 64
BB = 512


def _score_body(ue_ref, pe_ref, ne_ref, pos_ref, neg_ref, sq_ref):
    ue = ue_ref[...]
    pe = pe_ref[...]
    ne = ne_ref[...]
    pos_ref[...] = jnp.sum(ue * pe, axis=-1, keepdims=True)
    neg_ref[...] = jax.lax.dot_general(
        ne, ue,
        dimension_numbers=(((2,), (1,)), ((0,), (0,))),
        preferred_element_type=jnp.float32,
    )
    v = jnp.sum(ue * ue) + jnp.sum(pe * pe) + jnp.sum(ne * ne)

    @pl.when(pl.program_id(0) == 0)
    def _init():
        sq_ref[...] = jnp.zeros((1, 128), dtype=jnp.float32)

    sq_ref[...] += jnp.full((1, 128), v / 128.0, dtype=jnp.float32)


def kernel(user, item, item_negs, edge_u, edge_i, user_table, item_table):
    ones = jnp.ones(edge_u.shape[0], dtype=jnp.float32)
    deg_u = jax.ops.segment_sum(ones, edge_u, num_segments=NUM_USER)
    deg_i = jax.ops.segment_sum(ones, edge_i, num_segments=NUM_ITEM)
    w = jax.lax.rsqrt(deg_u[edge_u]) * jax.lax.rsqrt(deg_i[edge_i])
    users = [user_table]
    items = [item_table]
    for _ in range(NUM_LAYER):
        u_new = jax.ops.segment_sum(w[:, None] * items[-1][edge_i], edge_u,
                                    num_segments=NUM_USER)
        i_new = jax.ops.segment_sum(w[:, None] * users[-1][edge_u], edge_i,
                                    num_segments=NUM_ITEM)
        users.append(u_new)
        items.append(i_new)
    users_emb = jnp.stack(users, axis=1).mean(axis=1)
    items_emb = jnp.stack(items, axis=1).mean(axis=1)
    user_e = users_emb[user]
    pos_item_e = items_emb[item]
    neg_items_e = items_emb[item_negs]

    nblk = BATCH // BB
    pos, neg, sq = pl.pallas_call(
        _score_body,
        grid=(nblk,),
        in_specs=[
            pl.BlockSpec((BB, EMBED_DIM), lambda b: (b, 0)),
            pl.BlockSpec((BB, EMBED_DIM), lambda b: (b, 0)),
            pl.BlockSpec((BB, N_NEG, EMBED_DIM), lambda b: (b, 0, 0)),
        ],
        out_specs=[
            pl.BlockSpec((BB, 1), lambda b: (b, 0)),
            pl.BlockSpec((BB, N_NEG), lambda b: (b, 0)),
            pl.BlockSpec((1, 128), lambda b: (0, 0)),
        ],
        out_shape=[
            jax.ShapeDtypeStruct((BATCH, 1), jnp.float32),
            jax.ShapeDtypeStruct((BATCH, N_NEG), jnp.float32),
            jax.ShapeDtypeStruct((1, 128), jnp.float32),
        ],
    )(user_e, pos_item_e, neg_items_e)
    reg_loss = 0.5 * jnp.sum(sq) / float(BATCH)
    return pos, neg, reg_loss


# speedup vs baseline: 8.5689x; 8.5689x over previous
import functools

import jax
import jax.numpy as jnp
from jax import lax
from jax.experimental import pallas as pl
from jax.experimental.pallas import tpu as pltpu
from jax.experimental.pallas import tpu_sc as plsc

NUM_USER = 50000
NUM_ITEM = 50000
NUM_EDGE = 800000
EMBED_DIM = 64
HALF_DIM = 32
NUM_LAYER = 3
BATCH = 4096
N_NEG = 64

N_TILE = 16
N_CORE = 2
CHUNK = 128
BLK = 56
NBLK = 7
TILE_CHUNKS = BLK * NBLK
TILE_EDGES = TILE_CHUNKS * CHUNK
EDGE_PAD = N_TILE * TILE_EDGES
N_PAD = EDGE_PAD - NUM_EDGE
ACC_ROWS = 50048
ROWS_PER_TILE = ACC_ROWS // N_TILE
NQ = 17
QROWS = ROWS_PER_TILE // NQ

_MESH = plsc.VectorSubcoreMesh(core_axis_name="c", subcore_axis_name="s")
_SC_PARAMS = pltpu.CompilerParams(use_tc_tiling_on_sc=False)


def _deg_body(z1, du_idx, di_idx, deg_u, deg_i, acc, onesv, didxv, zstage):
    c = lax.axis_index("c")
    t = lax.axis_index("s")

    def fill_ones(i, _):
        onesv[pl.ds(i * 16, 16)] = jnp.ones((16,), jnp.float32)
        return _

    lax.fori_loop(0, CHUNK // 16, fill_ones, None)
    rpt = t * ROWS_PER_TILE
    pltpu.sync_copy(z1.at[pl.ds(rpt, ROWS_PER_TILE)], zstage)
    pltpu.sync_copy(zstage, acc.at[pl.ds(rpt, ROWS_PER_TILE)])
    plsc.subcore_barrier()

    def blk(b, _):
        crow = t * TILE_CHUNKS + b * BLK

        @pl.when(c == 0)
        def _():
            pltpu.sync_copy(du_idx.at[pl.ds(crow, BLK)], didxv)

        @pl.when(c == 1)
        def _():
            pltpu.sync_copy(di_idx.at[pl.ds(crow, BLK)], didxv)

        def chunk(j, _):
            pltpu.sync_copy(onesv, acc.at[didxv.at[j]], add=True)
            return _

        lax.fori_loop(0, BLK, chunk, None)
        return _

    lax.fori_loop(0, NBLK, blk, None)
    plsc.subcore_barrier()
    rb = t * ROWS_PER_TILE
    pltpu.sync_copy(acc.at[pl.ds(rb, ROWS_PER_TILE)], zstage)

    @pl.when(c == 0)
    def _():
        pltpu.sync_copy(zstage, deg_u.at[pl.ds(rb, ROWS_PER_TILE)])

    @pl.when(c == 1)
    def _():
        pltpu.sync_copy(zstage, deg_i.at[pl.ds(rb, ROWS_PER_TILE)])


_k_deg = pl.kernel(
    _deg_body,
    out_type=[jax.ShapeDtypeStruct((ACC_ROWS,), jnp.float32),
              jax.ShapeDtypeStruct((ACC_ROWS,), jnp.float32)],
    mesh=_MESH,
    compiler_params=_SC_PARAMS,
    scratch_types=[
        pltpu.VMEM_SHARED((ACC_ROWS,), jnp.float32),
        pltpu.VMEM((CHUNK,), jnp.float32),
        pltpu.VMEM((BLK, CHUNK), jnp.int32),
        pltpu.VMEM((ROWS_PER_TILE,), jnp.float32),
    ],
)


def _prop_body(s0, s1, z2, sidx, didx, d0, d1, acc, gbuf, sidxv, didxv,
               stage, sem):
    c = lax.axis_index("c")
    t = lax.axis_index("s")

    def zero_q(q, _):
        r = t * ROWS_PER_TILE + q * QROWS
        pltpu.sync_copy(z2.at[pl.ds(r, QROWS)], stage)
        pltpu.sync_copy(stage, acc.at[pl.ds(r, QROWS)])
        return _

    lax.fori_loop(0, NQ, zero_q, None)
    plsc.subcore_barrier()

    def blk(b, _):
        eoff = t * TILE_EDGES + b * (BLK * CHUNK)
        pltpu.sync_copy(sidx.at[pl.ds(eoff, BLK * CHUNK)], sidxv)
        crow = t * TILE_CHUNKS + b * BLK
        pltpu.sync_copy(didx.at[pl.ds(crow, BLK)], didxv)

        def chunk(j, _):
            islice = sidxv.at[pl.ds(j * CHUNK, CHUNK)]

            @pl.when(c == 0)
            def _():
                pltpu.async_copy(s0.at[islice], gbuf, sem).wait()

            @pl.when(c == 1)
            def _():
                pltpu.async_copy(s1.at[islice], gbuf, sem).wait()

            pltpu.sync_copy(gbuf, acc.at[didxv.at[j]], add=True)
            return _

        lax.fori_loop(0, BLK, chunk, None)
        return _

    lax.fori_loop(0, NBLK, blk, None)
    plsc.subcore_barrier()

    def wb_q(q, _):
        r = t * ROWS_PER_TILE + q * QROWS
        pltpu.sync_copy(acc.at[pl.ds(r, QROWS)], stage)

        @pl.when(c == 0)
        def _():
            pltpu.sync_copy(stage, d0.at[pl.ds(r, QROWS)])

        @pl.when(c == 1)
        def _():
            pltpu.sync_copy(stage, d1.at[pl.ds(r, QROWS)])

        return _

    lax.fori_loop(0, NQ, wb_q, None)


_k_prop = pl.kernel(
    _prop_body,
    out_type=[jax.ShapeDtypeStruct((ACC_ROWS, HALF_DIM), jnp.float32),
              jax.ShapeDtypeStruct((ACC_ROWS, HALF_DIM), jnp.float32)],
    mesh=_MESH,
    compiler_params=_SC_PARAMS,
    scratch_types=[
        pltpu.VMEM_SHARED((ACC_ROWS, HALF_DIM), jnp.float32),
        pltpu.VMEM((CHUNK, HALF_DIM), jnp.float32),
        pltpu.VMEM((BLK * CHUNK,), jnp.int32),
        pltpu.VMEM((BLK, CHUNK), jnp.int32),
        pltpu.VMEM((QROWS, HALF_DIM), jnp.float32),
        pltpu.SemaphoreType.DMA,
    ],
)


NEG_CHUNKS = BATCH * N_NEG // CHUNK
NEG_PER_W = NEG_CHUNKS // (N_TILE * N_CORE)
B_CHUNKS = BATCH // CHUNK


def _gather_body(uemb, iemb, uidx, iidx, nidx, ue, pe, ne,
                 gbuf, uidxv, nidxv, sem):
    c = lax.axis_index("c")
    s = lax.axis_index("s")
    w = s * N_CORE + c

    pltpu.sync_copy(uidx.at[pl.ds(w, 1)], uidxv)
    pltpu.async_copy(uemb.at[uidxv.at[0]], gbuf, sem).wait()
    pltpu.sync_copy(gbuf, ue.at[pl.ds(w * CHUNK, CHUNK)])
    pltpu.sync_copy(iidx.at[pl.ds(w, 1)], uidxv)
    pltpu.async_copy(iemb.at[uidxv.at[0]], gbuf, sem).wait()
    pltpu.sync_copy(gbuf, pe.at[pl.ds(w * CHUNK, CHUNK)])
    pltpu.sync_copy(nidx.at[pl.ds(w * NEG_PER_W, NEG_PER_W)], nidxv)

    def chunk(j, _):
        pltpu.async_copy(iemb.at[nidxv.at[j]], gbuf, sem).wait()
        pltpu.sync_copy(gbuf, ne.at[pl.ds((w * NEG_PER_W + j) * CHUNK, CHUNK)])
        return _

    lax.fori_loop(0, NEG_PER_W, chunk, None)


_k_gather = pl.kernel(
    _gather_body,
    out_type=[jax.ShapeDtypeStruct((BATCH, EMBED_DIM), jnp.float32),
              jax.ShapeDtypeStruct((BATCH, EMBED_DIM), jnp.float32),
              jax.ShapeDtypeStruct((BATCH * N_NEG, EMBED_DIM), jnp.float32)],
    mesh=_MESH,
    compiler_params=_SC_PARAMS,
    scratch_types=[
        pltpu.VMEM((CHUNK, EMBED_DIM), jnp.float32),
        pltpu.VMEM((1, CHUNK), jnp.int32),
        pltpu.VMEM((NEG_PER_W, CHUNK), jnp.int32),
        pltpu.SemaphoreType.DMA,
    ],
)


RB = 3128
N_RB = ACC_ROWS // RB


def _factors_body(deg_ref, inv_ref, ra_ref, fs_ref):
    d = jnp.maximum(deg_ref[...], 1.0)
    inv_ref[...] = 1.0 / d
    ra_ref[...] = lax.rsqrt(d)
    fs_ref[...] = 0.25 * jnp.sqrt(d)


def _factors(deg):
    return pl.pallas_call(
        _factors_body,
        grid=(N_RB,),
        in_specs=[pl.BlockSpec((RB, 1), lambda b: (b, 0))],
        out_specs=[pl.BlockSpec((RB, 1), lambda b: (b, 0))] * 3,
        out_shape=[jax.ShapeDtypeStruct((ACC_ROWS, 1), jnp.float32)] * 3,
    )(deg.reshape(ACC_ROWS, 1))


def _scale_split_body(t_ref, s_ref, h0_ref, h1_ref):
    h = t_ref[...] * s_ref[...]
    h0_ref[...] = h[:, :HALF_DIM]
    h1_ref[...] = h[:, HALF_DIM:]


def _scale_split(table, s):
    return pl.pallas_call(
        _scale_split_body,
        grid=(N_RB,),
        in_specs=[pl.BlockSpec((RB, EMBED_DIM), lambda b: (b, 0)),
                  pl.BlockSpec((RB, 1), lambda b: (b, 0))],
        out_specs=[pl.BlockSpec((RB, HALF_DIM), lambda b: (b, 0))] * 2,
        out_shape=[jax.ShapeDtypeStruct((ACC_ROWS, HALF_DIM), jnp.float32)] * 2,
    )(table, s)


def _scale2_body(h0_ref, h1_ref, s_ref, o0_ref, o1_ref):
    s = s_ref[...]
    o0_ref[...] = h0_ref[...] * s
    o1_ref[...] = h1_ref[...] * s


def _scale2(h0, h1, s):
    return pl.pallas_call(
        _scale2_body,
        grid=(N_RB,),
        in_specs=[pl.BlockSpec((RB, HALF_DIM), lambda b: (b, 0))] * 2
                 + [pl.BlockSpec((RB, 1), lambda b: (b, 0))],
        out_specs=[pl.BlockSpec((RB, HALF_DIM), lambda b: (b, 0))] * 2,
        out_shape=[jax.ShapeDtypeStruct((ACC_ROWS, HALF_DIM), jnp.float32)] * 2,
    )(h0, h1, s)


def _combine_body(a0, b0, c0, d0, a1, b1, c1, d1, s_ref, out_ref):
    s = s_ref[...]
    h0 = (a0[...] + b0[...] + c0[...] + d0[...]) * s
    h1 = (a1[...] + b1[...] + c1[...] + d1[...]) * s
    out_ref[...] = jnp.concatenate([h0, h1], axis=1)


def _combine(h0s, h1s, s):
    return pl.pallas_call(
        _combine_body,
        grid=(N_RB,),
        in_specs=[pl.BlockSpec((RB, HALF_DIM), lambda b: (b, 0))] * 8
                 + [pl.BlockSpec((RB, 1), lambda b: (b, 0))],
        out_specs=pl.BlockSpec((RB, EMBED_DIM), lambda b: (b, 0)),
        out_shape=jax.ShapeDtypeStruct((ACC_ROWS, EMBED_DIM), jnp.float32),
    )(*h0s, *h1s, s)


BB = 512


def _score_body(ue_ref, pe_ref, ne_ref, pos_ref, neg_ref, sq_ref):
    ue = ue_ref[...]
    pe = pe_ref[...]
    ne = ne_ref[...]
    pos_ref[...] = jnp.sum(ue * pe, axis=-1, keepdims=True)
    neg_ref[...] = lax.dot_general(
        ne, ue,
        dimension_numbers=(((2,), (1,)), ((0,), (0,))),
        preferred_element_type=jnp.float32,
    )
    v = jnp.sum(ue * ue) + jnp.sum(pe * pe) + jnp.sum(ne * ne)

    @pl.when(pl.program_id(0) == 0)
    def _init():
        sq_ref[...] = jnp.zeros((1, 128), dtype=jnp.float32)

    sq_ref[...] += jnp.full((1, 128), v / 128.0, dtype=jnp.float32)


def _score(ue, pe, ne):
    nblk = BATCH // BB
    return pl.pallas_call(
        _score_body,
        grid=(nblk,),
        in_specs=[
            pl.BlockSpec((BB, EMBED_DIM), lambda b: (b, 0)),
            pl.BlockSpec((BB, EMBED_DIM), lambda b: (b, 0)),
            pl.BlockSpec((BB, N_NEG, EMBED_DIM), lambda b: (b, 0, 0)),
        ],
        out_specs=[
            pl.BlockSpec((BB, 1), lambda b: (b, 0)),
            pl.BlockSpec((BB, N_NEG), lambda b: (b, 0)),
            pl.BlockSpec((1, 128), lambda b: (0, 0)),
        ],
        out_shape=[
            jax.ShapeDtypeStruct((BATCH, 1), jnp.float32),
            jax.ShapeDtypeStruct((BATCH, N_NEG), jnp.float32),
            jax.ShapeDtypeStruct((1, 128), jnp.float32),
        ],
    )(ue, pe, ne)


def kernel(user, item, item_negs, edge_u, edge_i, user_table, item_table):
    pad_ids = jnp.arange(N_PAD, dtype=jnp.int32)
    src_pad = (pad_ids * 97) % NUM_USER
    dst_pad = NUM_USER + pad_ids % (ACC_ROWS - NUM_USER)
    eu_src = jnp.concatenate([edge_u, src_pad])
    ei_src = jnp.concatenate([edge_i, src_pad])
    eu_dst = jnp.concatenate([edge_u, dst_pad]).reshape(EDGE_PAD // CHUNK, CHUNK)
    ei_dst = jnp.concatenate([edge_i, dst_pad]).reshape(EDGE_PAD // CHUNK, CHUNK)
    zeros1 = jnp.zeros((ACC_ROWS,), jnp.float32)
    zeros2 = jnp.zeros((ACC_ROWS, HALF_DIM), jnp.float32)

    deg_u, deg_i = _k_deg(zeros1, eu_dst, ei_dst)
    inv_u, ra_u, fs_u = _factors(deg_u)
    inv_i, ra_i, fs_i = _factors(deg_i)

    u_h0, u_h1 = [None] * 4, [None] * 4
    i_h0, i_h1 = [None] * 4, [None] * 4
    ut_pad = jnp.pad(user_table, ((0, ACC_ROWS - NUM_USER), (0, 0)))
    it_pad = jnp.pad(item_table, ((0, ACC_ROWS - NUM_ITEM), (0, 0)))
    u_h0[0], u_h1[0] = _scale_split(ut_pad, ra_u)
    i_h0[0], i_h1[0] = _scale_split(it_pad, ra_i)

    for k in range(NUM_LAYER):
        a0, a1 = _k_prop(i_h0[k], i_h1[k], zeros2, ei_src, eu_dst)
        u_h0[k + 1], u_h1[k + 1] = _scale2(a0, a1, inv_u)
        b0, b1 = _k_prop(u_h0[k], u_h1[k], zeros2, eu_src, ei_dst)
        i_h0[k + 1], i_h1[k + 1] = _scale2(b0, b1, inv_i)

    users_emb = _combine(u_h0, u_h1, fs_u)
    items_emb = _combine(i_h0, i_h1, fs_i)

    uidx = user.reshape(B_CHUNKS, CHUNK)
    iidx = item.reshape(B_CHUNKS, CHUNK)
    nidx = item_negs.reshape(NEG_CHUNKS, CHUNK)
    ue, pe, ne = _k_gather(users_emb, items_emb, uidx, iidx, nidx)

    pos, neg, sq = _score(ue, pe, ne.reshape(BATCH, N_NEG, EMBED_DIM))
    reg_loss = 0.5 * jnp.sum(sq) / float(BATCH)
    return pos, neg, reg_loss


# speedup vs baseline: 14.4998x; 1.6921x over previous
import functools

import jax
import jax.numpy as jnp
from jax import lax
from jax.experimental import pallas as pl
from jax.experimental.pallas import tpu as pltpu
from jax.experimental.pallas import tpu_sc as plsc

NUM_USER = 50000
NUM_ITEM = 50000
NUM_EDGE = 800000
EMBED_DIM = 64
HALF_DIM = 32
NUM_LAYER = 3
BATCH = 4096
N_NEG = 64

N_TILE = 16
N_CORE = 2
CHUNK = 128
BLK = 28
NBLK = 14
TILE_CHUNKS = BLK * NBLK
TILE_EDGES = TILE_CHUNKS * CHUNK
EDGE_PAD = N_TILE * TILE_EDGES
N_PAD = EDGE_PAD - NUM_EDGE
ACC_ROWS = 50048
ROWS_PER_TILE = ACC_ROWS // N_TILE
NRING = 4
NQ = 23
QROWS = ROWS_PER_TILE // NQ

_MESH = plsc.VectorSubcoreMesh(core_axis_name="c", subcore_axis_name="s")
_SC_PARAMS = pltpu.CompilerParams(use_tc_tiling_on_sc=False)


def _deg_body(z1, du_idx, di_idx, deg_u, deg_i, acc, onesv, didxv, zstage):
    c = lax.axis_index("c")
    t = lax.axis_index("s")

    def fill_ones(i, _):
        onesv[pl.ds(i * 16, 16)] = jnp.ones((16,), jnp.float32)
        return _

    lax.fori_loop(0, CHUNK // 16, fill_ones, None)
    rpt = t * ROWS_PER_TILE
    pltpu.sync_copy(z1.at[pl.ds(rpt, ROWS_PER_TILE)], zstage)
    pltpu.sync_copy(zstage, acc.at[pl.ds(rpt, ROWS_PER_TILE)])
    plsc.subcore_barrier()

    def blk(b, _):
        crow = t * TILE_CHUNKS + b * BLK

        @pl.when(c == 0)
        def _():
            pltpu.sync_copy(du_idx.at[pl.ds(crow, BLK)], didxv)

        @pl.when(c == 1)
        def _():
            pltpu.sync_copy(di_idx.at[pl.ds(crow, BLK)], didxv)

        def chunk(j, _):
            pltpu.sync_copy(onesv, acc.at[didxv.at[j]], add=True)
            return _

        lax.fori_loop(0, BLK, chunk, None)
        return _

    lax.fori_loop(0, NBLK, blk, None)
    plsc.subcore_barrier()
    rb = t * ROWS_PER_TILE
    pltpu.sync_copy(acc.at[pl.ds(rb, ROWS_PER_TILE)], zstage)

    @pl.when(c == 0)
    def _():
        pltpu.sync_copy(zstage, deg_u.at[pl.ds(rb, ROWS_PER_TILE)])

    @pl.when(c == 1)
    def _():
        pltpu.sync_copy(zstage, deg_i.at[pl.ds(rb, ROWS_PER_TILE)])


_k_deg = pl.kernel(
    _deg_body,
    out_type=[jax.ShapeDtypeStruct((ACC_ROWS,), jnp.float32),
              jax.ShapeDtypeStruct((ACC_ROWS,), jnp.float32)],
    mesh=_MESH,
    compiler_params=_SC_PARAMS,
    scratch_types=[
        pltpu.VMEM_SHARED((ACC_ROWS,), jnp.float32),
        pltpu.VMEM((CHUNK,), jnp.float32),
        pltpu.VMEM((BLK, CHUNK), jnp.int32),
        pltpu.VMEM((ROWS_PER_TILE,), jnp.float32),
    ],
)


def _prop_body(s0, s1, z2, sidx, didx, d0, d1, acc, gbuf, sidxv, didxv,
               stage, *sems):
    c = lax.axis_index("c")
    t = lax.axis_index("s")

    def zero_q(q, _):
        r = t * ROWS_PER_TILE + q * QROWS
        pltpu.sync_copy(z2.at[pl.ds(r, QROWS)], stage)
        pltpu.sync_copy(stage, acc.at[pl.ds(r, QROWS)])
        return _

    lax.fori_loop(0, NQ, zero_q, None)
    plsc.subcore_barrier()

    def fire(j, b):
        islice = sidxv.at[pl.ds(j * CHUNK, CHUNK)]
        dst = gbuf.at[pl.ds(b * CHUNK, CHUNK)]

        @pl.when(c == 0)
        def _():
            pltpu.async_copy(s0.at[islice], dst, sems[b])

        @pl.when(c == 1)
        def _():
            pltpu.async_copy(s1.at[islice], dst, sems[b])

    def blk(b, _):
        eoff = t * TILE_EDGES + b * (BLK * CHUNK)
        pltpu.sync_copy(sidx.at[pl.ds(eoff, BLK * CHUNK)], sidxv)
        crow = t * TILE_CHUNKS + b * BLK
        pltpu.sync_copy(didx.at[pl.ds(crow, BLK)], didxv)

        for q in range(NRING):
            fire(q, q)

        def group(g, _):
            for q in range(NRING):
                j = g * NRING + q
                gb = gbuf.at[pl.ds(q * CHUNK, CHUNK)]
                pltpu.make_async_copy(s0.at[sidxv.at[pl.ds(0, CHUNK)]],
                                      gb, sems[q]).wait()
                pltpu.sync_copy(gb, acc.at[didxv.at[j]], add=True)

                @pl.when(g < BLK // NRING - 1)
                def _():
                    fire(j + NRING, q)

            return _

        lax.fori_loop(0, BLK // NRING, group, None)
        return _

    lax.fori_loop(0, NBLK, blk, None)
    plsc.subcore_barrier()

    def wb_q(q, _):
        r = t * ROWS_PER_TILE + q * QROWS
        pltpu.sync_copy(acc.at[pl.ds(r, QROWS)], stage)

        @pl.when(c == 0)
        def _():
            pltpu.sync_copy(stage, d0.at[pl.ds(r, QROWS)])

        @pl.when(c == 1)
        def _():
            pltpu.sync_copy(stage, d1.at[pl.ds(r, QROWS)])

        return _

    lax.fori_loop(0, NQ, wb_q, None)


_k_prop = pl.kernel(
    _prop_body,
    out_type=[jax.ShapeDtypeStruct((ACC_ROWS, HALF_DIM), jnp.float32),
              jax.ShapeDtypeStruct((ACC_ROWS, HALF_DIM), jnp.float32)],
    mesh=_MESH,
    compiler_params=_SC_PARAMS,
    scratch_types=[
        pltpu.VMEM_SHARED((ACC_ROWS, HALF_DIM), jnp.float32),
        pltpu.VMEM((NRING * CHUNK, HALF_DIM), jnp.float32),
        pltpu.VMEM((BLK * CHUNK,), jnp.int32),
        pltpu.VMEM((BLK, CHUNK), jnp.int32),
        pltpu.VMEM((QROWS, HALF_DIM), jnp.float32),
    ] + [pltpu.SemaphoreType.DMA] * NRING,
)


NEG_CHUNKS = BATCH * N_NEG // CHUNK
NEG_PER_W = NEG_CHUNKS // (N_TILE * N_CORE)
B_CHUNKS = BATCH // CHUNK


def _gather_body(uemb, iemb, uidx, iidx, nidx, ue, pe, ne,
                 gbuf, uidxv, nidxv, sem):
    c = lax.axis_index("c")
    s = lax.axis_index("s")
    w = s * N_CORE + c

    pltpu.sync_copy(uidx.at[pl.ds(w, 1)], uidxv)
    pltpu.async_copy(uemb.at[uidxv.at[0]], gbuf, sem).wait()
    pltpu.sync_copy(gbuf, ue.at[pl.ds(w * CHUNK, CHUNK)])
    pltpu.sync_copy(iidx.at[pl.ds(w, 1)], uidxv)
    pltpu.async_copy(iemb.at[uidxv.at[0]], gbuf, sem).wait()
    pltpu.sync_copy(gbuf, pe.at[pl.ds(w * CHUNK, CHUNK)])
    pltpu.sync_copy(nidx.at[pl.ds(w * NEG_PER_W, NEG_PER_W)], nidxv)

    def chunk(j, _):
        pltpu.async_copy(iemb.at[nidxv.at[j]], gbuf, sem).wait()
        pltpu.sync_copy(gbuf, ne.at[pl.ds((w * NEG_PER_W + j) * CHUNK, CHUNK)])
        return _

    lax.fori_loop(0, NEG_PER_W, chunk, None)


_k_gather = pl.kernel(
    _gather_body,
    out_type=[jax.ShapeDtypeStruct((BATCH, EMBED_DIM), jnp.float32),
              jax.ShapeDtypeStruct((BATCH, EMBED_DIM), jnp.float32),
              jax.ShapeDtypeStruct((BATCH * N_NEG, EMBED_DIM), jnp.float32)],
    mesh=_MESH,
    compiler_params=_SC_PARAMS,
    scratch_types=[
        pltpu.VMEM((CHUNK, EMBED_DIM), jnp.float32),
        pltpu.VMEM((1, CHUNK), jnp.int32),
        pltpu.VMEM((NEG_PER_W, CHUNK), jnp.int32),
        pltpu.SemaphoreType.DMA,
    ],
)


RB = 3128
N_RB = ACC_ROWS // RB


def _factors_body(deg_ref, inv_ref, ra_ref, fs_ref):
    d = jnp.maximum(deg_ref[...], 1.0)
    inv_ref[...] = 1.0 / d
    ra_ref[...] = lax.rsqrt(d)
    fs_ref[...] = 0.25 * jnp.sqrt(d)


def _factors(deg):
    return pl.pallas_call(
        _factors_body,
        grid=(N_RB,),
        in_specs=[pl.BlockSpec((RB, 1), lambda b: (b, 0))],
        out_specs=[pl.BlockSpec((RB, 1), lambda b: (b, 0))] * 3,
        out_shape=[jax.ShapeDtypeStruct((ACC_ROWS, 1), jnp.float32)] * 3,
    )(deg.reshape(ACC_ROWS, 1))


def _scale_split_body(t_ref, s_ref, h0_ref, h1_ref):
    h = t_ref[...] * s_ref[...]
    h0_ref[...] = h[:, :HALF_DIM]
    h1_ref[...] = h[:, HALF_DIM:]


def _scale_split(table, s):
    return pl.pallas_call(
        _scale_split_body,
        grid=(N_RB,),
        in_specs=[pl.BlockSpec((RB, EMBED_DIM), lambda b: (b, 0)),
                  pl.BlockSpec((RB, 1), lambda b: (b, 0))],
        out_specs=[pl.BlockSpec((RB, HALF_DIM), lambda b: (b, 0))] * 2,
        out_shape=[jax.ShapeDtypeStruct((ACC_ROWS, HALF_DIM), jnp.float32)] * 2,
    )(table, s)


def _scale2_body(h0_ref, h1_ref, s_ref, o0_ref, o1_ref):
    s = s_ref[...]
    o0_ref[...] = h0_ref[...] * s
    o1_ref[...] = h1_ref[...] * s


def _scale2(h0, h1, s):
    return pl.pallas_call(
        _scale2_body,
        grid=(N_RB,),
        in_specs=[pl.BlockSpec((RB, HALF_DIM), lambda b: (b, 0))] * 2
                 + [pl.BlockSpec((RB, 1), lambda b: (b, 0))],
        out_specs=[pl.BlockSpec((RB, HALF_DIM), lambda b: (b, 0))] * 2,
        out_shape=[jax.ShapeDtypeStruct((ACC_ROWS, HALF_DIM), jnp.float32)] * 2,
    )(h0, h1, s)


def _combine_body(a0, b0, c0, d0, a1, b1, c1, d1, s_ref, out_ref):
    s = s_ref[...]
    h0 = (a0[...] + b0[...] + c0[...] + d0[...]) * s
    h1 = (a1[...] + b1[...] + c1[...] + d1[...]) * s
    out_ref[...] = jnp.concatenate([h0, h1], axis=1)


def _combine(h0s, h1s, s):
    return pl.pallas_call(
        _combine_body,
        grid=(N_RB,),
        in_specs=[pl.BlockSpec((RB, HALF_DIM), lambda b: (b, 0))] * 8
                 + [pl.BlockSpec((RB, 1), lambda b: (b, 0))],
        out_specs=pl.BlockSpec((RB, EMBED_DIM), lambda b: (b, 0)),
        out_shape=jax.ShapeDtypeStruct((ACC_ROWS, EMBED_DIM), jnp.float32),
    )(*h0s, *h1s, s)


BB = 512


def _score_body(ue_ref, pe_ref, ne_ref, pos_ref, neg_ref, sq_ref):
    ue = ue_ref[...]
    pe = pe_ref[...]
    ne = ne_ref[...]
    pos_ref[...] = jnp.sum(ue * pe, axis=-1, keepdims=True)
    neg_ref[...] = lax.dot_general(
        ne, ue,
        dimension_numbers=(((2,), (1,)), ((0,), (0,))),
        preferred_element_type=jnp.float32,
    )
    v = jnp.sum(ue * ue) + jnp.sum(pe * pe) + jnp.sum(ne * ne)

    @pl.when(pl.program_id(0) == 0)
    def _init():
        sq_ref[...] = jnp.zeros((1, 128), dtype=jnp.float32)

    sq_ref[...] += jnp.full((1, 128), v / 128.0, dtype=jnp.float32)


def _score(ue, pe, ne):
    nblk = BATCH // BB
    return pl.pallas_call(
        _score_body,
        grid=(nblk,),
        in_specs=[
            pl.BlockSpec((BB, EMBED_DIM), lambda b: (b, 0)),
            pl.BlockSpec((BB, EMBED_DIM), lambda b: (b, 0)),
            pl.BlockSpec((BB, N_NEG, EMBED_DIM), lambda b: (b, 0, 0)),
        ],
        out_specs=[
            pl.BlockSpec((BB, 1), lambda b: (b, 0)),
            pl.BlockSpec((BB, N_NEG), lambda b: (b, 0)),
            pl.BlockSpec((1, 128), lambda b: (0, 0)),
        ],
        out_shape=[
            jax.ShapeDtypeStruct((BATCH, 1), jnp.float32),
            jax.ShapeDtypeStruct((BATCH, N_NEG), jnp.float32),
            jax.ShapeDtypeStruct((1, 128), jnp.float32),
        ],
    )(ue, pe, ne)


def kernel(user, item, item_negs, edge_u, edge_i, user_table, item_table):
    pad_ids = jnp.arange(N_PAD, dtype=jnp.int32)
    src_pad = (pad_ids * 97) % NUM_USER
    dst_pad = NUM_USER + pad_ids % (ACC_ROWS - NUM_USER)
    eu_src = jnp.concatenate([edge_u, src_pad])
    ei_src = jnp.concatenate([edge_i, src_pad])
    eu_dst = jnp.concatenate([edge_u, dst_pad]).reshape(EDGE_PAD // CHUNK, CHUNK)
    ei_dst = jnp.concatenate([edge_i, dst_pad]).reshape(EDGE_PAD // CHUNK, CHUNK)
    zeros1 = jnp.zeros((ACC_ROWS,), jnp.float32)
    zeros2 = jnp.zeros((ACC_ROWS, HALF_DIM), jnp.float32)

    deg_u, deg_i = _k_deg(zeros1, eu_dst, ei_dst)
    inv_u, ra_u, fs_u = _factors(deg_u)
    inv_i, ra_i, fs_i = _factors(deg_i)

    u_h0, u_h1 = [None] * 4, [None] * 4
    i_h0, i_h1 = [None] * 4, [None] * 4
    ut_pad = jnp.pad(user_table, ((0, ACC_ROWS - NUM_USER), (0, 0)))
    it_pad = jnp.pad(item_table, ((0, ACC_ROWS - NUM_ITEM), (0, 0)))
    u_h0[0], u_h1[0] = _scale_split(ut_pad, ra_u)
    i_h0[0], i_h1[0] = _scale_split(it_pad, ra_i)

    for k in range(NUM_LAYER):
        a0, a1 = _k_prop(i_h0[k], i_h1[k], zeros2, ei_src, eu_dst)
        u_h0[k + 1], u_h1[k + 1] = _scale2(a0, a1, inv_u)
        b0, b1 = _k_prop(u_h0[k], u_h1[k], zeros2, eu_src, ei_dst)
        i_h0[k + 1], i_h1[k + 1] = _scale2(b0, b1, inv_i)

    users_emb = _combine(u_h0, u_h1, fs_u)
    items_emb = _combine(i_h0, i_h1, fs_i)

    uidx = user.reshape(B_CHUNKS, CHUNK)
    iidx = item.reshape(B_CHUNKS, CHUNK)
    nidx = item_negs.reshape(NEG_CHUNKS, CHUNK)
    ue, pe, ne = _k_gather(users_emb, items_emb, uidx, iidx, nidx)

    pos, neg, sq = _score(ue, pe, ne.reshape(BATCH, N_NEG, EMBED_DIM))
    reg_loss = 0.5 * jnp.sum(sq) / float(BATCH)
    return pos, neg, reg_loss
